# baseline (speedup 1.0000x reference)
import functools

import jax
import jax.numpy as jnp
from jax import lax
from jax.experimental import pallas as pl
from jax.experimental.pallas import tpu as pltpu
from jax.experimental.pallas import tpu_sc as plsc

B = 4096
R = 200
DF = 128
DR = 128
NUM_F = 100000
NB = B * R

NSPLIT = 4
NB_CALL = NB // NSPLIT
B_CALL = B // NSPLIT

NC = 2
NS = 16
NW = NC * NS
ROWS_PER_W = NB_CALL // NW
CHUNK = 128
N_CHUNKS = ROWS_PER_W // CHUNK
NBUF = 4


def _gather_sc(filler_emb, idx_flat):
    mesh = plsc.VectorSubcoreMesh(core_axis_name="c", subcore_axis_name="s")
    dt = filler_emb.dtype
    ncol = filler_emb.shape[1]

    @functools.partial(
        pl.kernel,
        mesh=mesh,
        out_type=jax.ShapeDtypeStruct((NB_CALL, ncol), dt),
        scratch_types=[
            pltpu.VMEM((ROWS_PER_W,), jnp.int32),
            pltpu.VMEM((NBUF, CHUNK, ncol), dt),
            [pltpu.SemaphoreType.DMA] * NBUF,
            [pltpu.SemaphoreType.DMA] * NBUF,
        ],
    )
    def k(table_hbm, idx_hbm, out_hbm, idx_v, rows_v, gsems, wsems):
        wid = lax.axis_index("s") * NC + lax.axis_index("c")
        base = wid * ROWS_PER_W
        pltpu.sync_copy(idx_hbm.at[pl.ds(base, ROWS_PER_W)], idx_v)

        def gather(j):
            pltpu.async_copy(
                table_hbm.at[idx_v.at[pl.ds(j * CHUNK, CHUNK)]],
                rows_v.at[j % NBUF],
                gsems[j % NBUF],
            )

        def gwait(j):
            pltpu.make_async_copy(
                table_hbm.at[idx_v.at[pl.ds(j * CHUNK, CHUNK)]],
                rows_v.at[j % NBUF],
                gsems[j % NBUF],
            ).wait()

        def wstart(j):
            pltpu.async_copy(
                rows_v.at[j % NBUF],
                out_hbm.at[pl.ds(base + j * CHUNK, CHUNK)],
                wsems[j % NBUF],
            )

        def wwait(j):
            pltpu.make_async_copy(
                rows_v.at[j % NBUF],
                out_hbm.at[pl.ds(base + j * CHUNK, CHUNK)],
                wsems[j % NBUF],
            ).wait()

        for j in range(min(NBUF - 1, N_CHUNKS)):
            gather(j)
        for j in range(N_CHUNKS):
            gwait(j)
            wstart(j)
            nxt = j + NBUF - 1
            if nxt < N_CHUNKS:
                if j >= 1:
                    wwait(j - 1)
                gather(nxt)
        for j in range(max(0, N_CHUNKS - NBUF), N_CHUNKS):
            wwait(j)

    return k(filler_emb, idx_flat)


BB = 8


def _mm_body_first(x_ref, role_ref, out_ref):
    role = role_ref[...]
    for i in range(BB):
        out_ref[i] = lax.dot_general(
            x_ref[i].astype(jnp.bfloat16),
            role,
            (((0,), (0,)), ((), ())),
            preferred_element_type=jnp.float32,
        )


def _mm_body(x_ref, role_ref, _prev_ref, out_ref):
    _mm_body_first(x_ref, role_ref, out_ref)


def _tpr_tc(x, role_emb, prev_out, b_off):
    if prev_out is None:
        return pl.pallas_call(
            _mm_body_first,
            grid=(B_CALL // BB,),
            in_specs=[
                pl.BlockSpec((BB, R, DF), lambda i: (i, 0, 0)),
                pl.BlockSpec((R, DR), lambda i: (0, 0)),
            ],
            out_specs=pl.BlockSpec((BB, DF, DR), lambda i: (i, 0, 0)),
            out_shape=jax.ShapeDtypeStruct((B, DF, DR), jnp.float32),
        )(x, role_emb)
    return pl.pallas_call(
        _mm_body,
        grid=(B_CALL // BB,),
        in_specs=[
            pl.BlockSpec((BB, R, DF), lambda i: (i, 0, 0)),
            pl.BlockSpec((R, DR), lambda i: (0, 0)),
            pl.BlockSpec(memory_space=pl.ANY),
        ],
        out_specs=pl.BlockSpec(
            (BB, DF, DR), lambda i, _o=b_off // BB: (i + _o, 0, 0)
        ),
        out_shape=jax.ShapeDtypeStruct((B, DF, DR), jnp.float32),
        input_output_aliases={2: 0},
    )(x, role_emb, prev_out)


def kernel(tree_tensor, filler_emb, role_emb):
    idx_flat = tree_tensor.reshape(-1)
    role_bf = role_emb.astype(jnp.bfloat16)
    out = None
    for s in range(NSPLIT):
        xs = _gather_sc(filler_emb, idx_flat[s * NB_CALL:(s + 1) * NB_CALL])
        out = _tpr_tc(xs.reshape(B_CALL, R, DF), role_bf, out, s * B_CALL)
    return out


# speedup vs baseline: 1.2105x; 1.2105x over previous
import functools

import jax
import jax.numpy as jnp
from jax import lax
from jax.experimental import pallas as pl
from jax.experimental.pallas import tpu as pltpu
from jax.experimental.pallas import tpu_sc as plsc

B = 4096
R = 200
DF = 128
DR = 128
NUM_F = 100000
NB = B * R

NSPLIT = 4
NB_CALL = NB // NSPLIT
B_CALL = B // NSPLIT

NC = 2
NS = 16
NW = NC * NS
ROWS_PER_W = NB_CALL // NW
CHUNK = 128
N_CHUNKS = ROWS_PER_W // CHUNK
NBUF = 4


def _gather_sc(filler_emb, idx_flat):
    mesh = plsc.VectorSubcoreMesh(core_axis_name="c", subcore_axis_name="s")
    dt = filler_emb.dtype
    ncol = filler_emb.shape[1]

    @functools.partial(
        pl.kernel,
        mesh=mesh,
        out_type=jax.ShapeDtypeStruct((NB_CALL, ncol), dt),
        scratch_types=[
            pltpu.VMEM((ROWS_PER_W,), jnp.int32),
            pltpu.VMEM((NBUF, CHUNK, ncol), dt),
            [pltpu.SemaphoreType.DMA] * NBUF,
            [pltpu.SemaphoreType.DMA] * NBUF,
        ],
    )
    def k(table_hbm, idx_hbm, out_hbm, idx_v, rows_v, gsems, wsems):
        wid = lax.axis_index("s") * NC + lax.axis_index("c")
        base = wid * ROWS_PER_W
        pltpu.sync_copy(idx_hbm.at[pl.ds(base, ROWS_PER_W)], idx_v)

        def gather(j):
            pltpu.async_copy(
                table_hbm.at[idx_v.at[pl.ds(j * CHUNK, CHUNK)]],
                rows_v.at[j % NBUF],
                gsems[j % NBUF],
            )

        def gwait(j):
            pltpu.make_async_copy(
                table_hbm.at[idx_v.at[pl.ds(j * CHUNK, CHUNK)]],
                rows_v.at[j % NBUF],
                gsems[j % NBUF],
            ).wait()

        def wstart(j):
            pltpu.async_copy(
                rows_v.at[j % NBUF],
                out_hbm.at[pl.ds(base + j * CHUNK, CHUNK)],
                wsems[j % NBUF],
            )

        def wwait(j):
            pltpu.make_async_copy(
                rows_v.at[j % NBUF],
                out_hbm.at[pl.ds(base + j * CHUNK, CHUNK)],
                wsems[j % NBUF],
            ).wait()

        for j in range(min(NBUF - 1, N_CHUNKS)):
            gather(j)
        for j in range(N_CHUNKS):
            gwait(j)
            wstart(j)
            nxt = j + NBUF - 1
            if nxt < N_CHUNKS:
                if j >= 1:
                    wwait(j - 1)
                gather(nxt)
        for j in range(max(0, N_CHUNKS - NBUF), N_CHUNKS):
            wwait(j)

    return k(filler_emb, idx_flat)


BB = 16


def _mm_body_first(x_ref, role_ref, out_ref):
    role = role_ref[...]
    for i in range(BB):
        out_ref[i] = lax.dot_general(
            x_ref[i].astype(jnp.bfloat16),
            role,
            (((0,), (0,)), ((), ())),
            preferred_element_type=jnp.float32,
        )


def _mm_body(x_ref, role_ref, _prev_ref, out_ref):
    _mm_body_first(x_ref, role_ref, out_ref)


def _tpr_tc(x, role_emb, prev_out, b_off):
    if prev_out is None:
        return pl.pallas_call(
            _mm_body_first,
            grid=(B_CALL // BB,),
            in_specs=[
                pl.BlockSpec((BB, R, DF), lambda i: (i, 0, 0)),
                pl.BlockSpec((R, DR), lambda i: (0, 0)),
            ],
            out_specs=pl.BlockSpec((BB, DF, DR), lambda i: (i, 0, 0)),
            out_shape=jax.ShapeDtypeStruct((B, DF, DR), jnp.float32),
        )(x, role_emb)
    return pl.pallas_call(
        _mm_body,
        grid=(B_CALL // BB,),
        in_specs=[
            pl.BlockSpec((BB, R, DF), lambda i: (i, 0, 0)),
            pl.BlockSpec((R, DR), lambda i: (0, 0)),
            pl.BlockSpec(memory_space=pl.ANY),
        ],
        out_specs=pl.BlockSpec(
            (BB, DF, DR), lambda i, _o=b_off // BB: (i + _o, 0, 0)
        ),
        out_shape=jax.ShapeDtypeStruct((B, DF, DR), jnp.float32),
        input_output_aliases={2: 0},
    )(x, role_emb, prev_out)


def kernel(tree_tensor, filler_emb, role_emb):
    idx_flat = tree_tensor.reshape(-1)
    role_bf = role_emb.astype(jnp.bfloat16)
    out = None
    for s in range(NSPLIT):
        xs = _gather_sc(filler_emb, idx_flat[s * NB_CALL:(s + 1) * NB_CALL])
        out = _tpr_tc(xs.reshape(B_CALL, R, DF), role_bf, out, s * B_CALL)
    return out


# speedup vs baseline: 1.2299x; 1.0160x over previous
import functools

import jax
import jax.numpy as jnp
from jax import lax
from jax.experimental import pallas as pl
from jax.experimental.pallas import tpu as pltpu
from jax.experimental.pallas import tpu_sc as plsc

B = 4096
R = 200
DF = 128
DR = 128
NUM_F = 100000
NB = B * R

NSPLIT = 8
NB_CALL = NB // NSPLIT
B_CALL = B // NSPLIT

NC = 2
NS = 16
NW = NC * NS
ROWS_PER_W = NB_CALL // NW
CHUNK = 128
N_CHUNKS = ROWS_PER_W // CHUNK
NBUF = 4


def _gather_sc(filler_emb, idx_flat):
    mesh = plsc.VectorSubcoreMesh(core_axis_name="c", subcore_axis_name="s")
    dt = filler_emb.dtype
    ncol = filler_emb.shape[1]

    @functools.partial(
        pl.kernel,
        mesh=mesh,
        out_type=jax.ShapeDtypeStruct((NB_CALL, ncol), dt),
        scratch_types=[
            pltpu.VMEM((ROWS_PER_W,), jnp.int32),
            pltpu.VMEM((NBUF, CHUNK, ncol), dt),
            [pltpu.SemaphoreType.DMA] * NBUF,
            [pltpu.SemaphoreType.DMA] * NBUF,
        ],
    )
    def k(table_hbm, idx_hbm, out_hbm, idx_v, rows_v, gsems, wsems):
        wid = lax.axis_index("s") * NC + lax.axis_index("c")
        base = wid * ROWS_PER_W
        pltpu.sync_copy(idx_hbm.at[pl.ds(base, ROWS_PER_W)], idx_v)

        def gather(j):
            pltpu.async_copy(
                table_hbm.at[idx_v.at[pl.ds(j * CHUNK, CHUNK)]],
                rows_v.at[j % NBUF],
                gsems[j % NBUF],
            )

        def gwait(j):
            pltpu.make_async_copy(
                table_hbm.at[idx_v.at[pl.ds(j * CHUNK, CHUNK)]],
                rows_v.at[j % NBUF],
                gsems[j % NBUF],
            ).wait()

        def wstart(j):
            pltpu.async_copy(
                rows_v.at[j % NBUF],
                out_hbm.at[pl.ds(base + j * CHUNK, CHUNK)],
                wsems[j % NBUF],
            )

        def wwait(j):
            pltpu.make_async_copy(
                rows_v.at[j % NBUF],
                out_hbm.at[pl.ds(base + j * CHUNK, CHUNK)],
                wsems[j % NBUF],
            ).wait()

        for j in range(min(NBUF - 1, N_CHUNKS)):
            gather(j)
        for j in range(N_CHUNKS):
            gwait(j)
            wstart(j)
            nxt = j + NBUF - 1
            if nxt < N_CHUNKS:
                if j >= 1:
                    wwait(j - 1)
                gather(nxt)
        for j in range(max(0, N_CHUNKS - NBUF), N_CHUNKS):
            wwait(j)

    return k(filler_emb, idx_flat)


BB = 16


def _mm_body_first(x_ref, role_ref, out_ref):
    role = role_ref[...]
    for i in range(BB):
        out_ref[i] = lax.dot_general(
            x_ref[i].astype(jnp.bfloat16),
            role,
            (((0,), (0,)), ((), ())),
            preferred_element_type=jnp.float32,
        )


def _mm_body(x_ref, role_ref, _prev_ref, out_ref):
    _mm_body_first(x_ref, role_ref, out_ref)


def _tpr_tc(x, role_emb, prev_out, b_off):
    if prev_out is None:
        return pl.pallas_call(
            _mm_body_first,
            grid=(B_CALL // BB,),
            in_specs=[
                pl.BlockSpec((BB, R, DF), lambda i: (i, 0, 0)),
                pl.BlockSpec((R, DR), lambda i: (0, 0)),
            ],
            out_specs=pl.BlockSpec((BB, DF, DR), lambda i: (i, 0, 0)),
            out_shape=jax.ShapeDtypeStruct((B, DF, DR), jnp.float32),
        )(x, role_emb)
    return pl.pallas_call(
        _mm_body,
        grid=(B_CALL // BB,),
        in_specs=[
            pl.BlockSpec((BB, R, DF), lambda i: (i, 0, 0)),
            pl.BlockSpec((R, DR), lambda i: (0, 0)),
            pl.BlockSpec(memory_space=pl.ANY),
        ],
        out_specs=pl.BlockSpec(
            (BB, DF, DR), lambda i, _o=b_off // BB: (i + _o, 0, 0)
        ),
        out_shape=jax.ShapeDtypeStruct((B, DF, DR), jnp.float32),
        input_output_aliases={2: 0},
    )(x, role_emb, prev_out)


def kernel(tree_tensor, filler_emb, role_emb):
    idx_flat = tree_tensor.reshape(-1)
    role_bf = role_emb.astype(jnp.bfloat16)
    out = None
    for s in range(NSPLIT):
        xs = _gather_sc(filler_emb, idx_flat[s * NB_CALL:(s + 1) * NB_CALL])
        out = _tpr_tc(xs.reshape(B_CALL, R, DF), role_bf, out, s * B_CALL)
    return out
